# Initial kernel scaffold; baseline (speedup 1.0000x reference)
import functools

import jax
import jax.numpy as jnp
from jax import lax
from jax.experimental import pallas as pl
from jax.experimental.pallas import tpu as pltpu
from jax.experimental.pallas import tpu_sc as plsc

B = 16384
K = 512
D = 512
W_ORI = 0.7
W_MAP = 0.3


TC_BLK = 512
TC_GRID = B // TC_BLK


def _tc_argmin_body(z_ref, p_ref, idx_ref):
    z = z_ref[...]
    p = p_ref[...]
    scores = lax.dot_general(
        z, p, dimension_numbers=(((1,), (1,)), ((), ())),
        preferred_element_type=jnp.float32)
    pn = jnp.sum(p * p, axis=1)
    dists = pn[None, :] - 2.0 * scores
    m = jnp.min(dists, axis=1, keepdims=True)
    iota = lax.broadcasted_iota(jnp.int32, dists.shape, 1)
    cand = jnp.where(dists == m, iota, K)
    idx_ref[0, 0, :] = jnp.min(cand, axis=1)


def _tc_argmin(z, p):
    out = pl.pallas_call(
        _tc_argmin_body,
        grid=(TC_GRID,),
        in_specs=[
            pl.BlockSpec((TC_BLK, D), lambda i: (i, 0)),
            pl.BlockSpec((K, D), lambda i: (0, 0)),
        ],
        out_specs=pl.BlockSpec((1, 1, TC_BLK), lambda i: (i, 0, 0)),
        out_shape=jax.ShapeDtypeStruct((TC_GRID, 1, TC_BLK), jnp.int32),
    )(z, p)
    return out.reshape(B)


NC = 2
NS = 16
NW = NC * NS
ROWS_W = B // NW
CH = 64
NCH = ROWS_W // CH

_SC_MESH = plsc.VectorSubcoreMesh(core_axis_name="c", subcore_axis_name="s")


@functools.partial(
    pl.kernel,
    out_type=(
        jax.ShapeDtypeStruct((B, D), jnp.float32),
        jax.ShapeDtypeStruct((B, D), jnp.float32),
    ),
    mesh=_SC_MESH,
    scratch_types=[
        pltpu.VMEM((CH,), jnp.int32),
        pltpu.VMEM((CH, D), jnp.float32),
        pltpu.VMEM((CH, D), jnp.float32),
        pltpu.VMEM((CH, D), jnp.float32),
        pltpu.SemaphoreType.DMA,
        pltpu.SemaphoreType.DMA,
    ],
)
def _sc_gather_blend(z_hbm, p_hbm, idx_hbm, zh_hbm, zt_hbm,
                     idx_v, rows_v, z_v, hat_v, sem_r, sem_z):
    wid = lax.axis_index("s") * NC + lax.axis_index("c")
    base = wid * ROWS_W

    def chunk(ci, carry):
        cb = base + ci * CH
        pltpu.sync_copy(idx_hbm.at[pl.ds(cb, CH)], idx_v)
        cp_r = pltpu.async_copy(p_hbm.at[idx_v], rows_v, sem_r)
        cp_z = pltpu.async_copy(z_hbm.at[pl.ds(cb, CH)], z_v, sem_z)
        cp_r.wait()
        pltpu.sync_copy(rows_v, zt_hbm.at[pl.ds(cb, CH)])
        cp_z.wait()

        def row(r, c2):
            for cc in range(D // 16):
                sl = pl.ds(cc * 16, 16)
                hat_v[r, sl] = W_ORI * z_v[r, sl] + W_MAP * rows_v[r, sl]
            return c2

        lax.fori_loop(0, CH, row, 0)
        pltpu.sync_copy(hat_v, zh_hbm.at[pl.ds(cb, CH)])
        return carry

    lax.fori_loop(0, NCH, chunk, 0)


def kernel(z, P):
    idx = _tc_argmin(z, P)
    z_hat, z_tilde = _sc_gather_blend(z, P, idx)
    return (z_hat, z_tilde, idx)


# speedup vs baseline: 1.0848x; 1.0848x over previous
import functools

import jax
import jax.numpy as jnp
from jax import lax
from jax.experimental import pallas as pl
from jax.experimental.pallas import tpu as pltpu
from jax.experimental.pallas import tpu_sc as plsc

B = 16384
K = 512
D = 512
W_ORI = 0.7
W_MAP = 0.3


TC_BLK = 512
TC_GRID = B // TC_BLK


def _tc_argmin_body(z_ref, p_ref, pn_ref, idx_ref):
    z = z_ref[...]
    p = p_ref[...]
    scores = lax.dot_general(
        z, p, dimension_numbers=(((1,), (1,)), ((), ())),
        preferred_element_type=jnp.float32)
    zn = jnp.sum(z * z, axis=1, keepdims=True)
    norms = zn + pn_ref[...]
    dists = norms - 2.0 * scores
    m = jnp.min(dists, axis=1, keepdims=True)
    iota = lax.broadcasted_iota(jnp.int32, dists.shape, 1)
    cand = jnp.where(dists == m, iota, K)
    idx_ref[...] = jnp.min(cand, axis=1, keepdims=True)


def _tc_argmin(z, p):
    pn = jnp.sum(p * p, axis=1)[None, :]
    out = pl.pallas_call(
        _tc_argmin_body,
        grid=(TC_GRID,),
        in_specs=[
            pl.BlockSpec((TC_BLK, D), lambda i: (i, 0)),
            pl.BlockSpec((K, D), lambda i: (0, 0)),
            pl.BlockSpec((1, K), lambda i: (0, 0)),
        ],
        out_specs=pl.BlockSpec((TC_BLK, 1), lambda i: (i, 0)),
        out_shape=jax.ShapeDtypeStruct((B, 1), jnp.int32),
    )(z, p, pn)
    return out.reshape(B)


NC = 2
NS = 16
NW = NC * NS
ROWS_W = B // NW
CH = 64
NCH = ROWS_W // CH

@functools.cache
def _sc_gather_blend():
    mesh = plsc.VectorSubcoreMesh(core_axis_name="c", subcore_axis_name="s")

    @functools.partial(
        pl.kernel,
        out_type=(
            jax.ShapeDtypeStruct((B, D), jnp.float32),
            jax.ShapeDtypeStruct((B, D), jnp.float32),
        ),
        mesh=mesh,
        scratch_types=[
            pltpu.VMEM((CH,), jnp.int32),
            pltpu.VMEM((CH, D), jnp.float32),
            pltpu.VMEM((CH, D), jnp.float32),
            pltpu.VMEM((CH, D), jnp.float32),
            pltpu.SemaphoreType.DMA,
            pltpu.SemaphoreType.DMA,
        ],
    )
    def body(z_hbm, p_hbm, idx_hbm, zh_hbm, zt_hbm,
             idx_v, rows_v, z_v, hat_v, sem_r, sem_z):
        wid = lax.axis_index("s") * NC + lax.axis_index("c")
        base = wid * ROWS_W

        def chunk(ci, carry):
            cb = base + ci * CH
            pltpu.sync_copy(idx_hbm.at[pl.ds(cb, CH)], idx_v)
            cp_r = pltpu.async_copy(p_hbm.at[idx_v], rows_v, sem_r)
            cp_z = pltpu.async_copy(z_hbm.at[pl.ds(cb, CH)], z_v, sem_z)
            cp_r.wait()
            pltpu.sync_copy(rows_v, zt_hbm.at[pl.ds(cb, CH)])
            cp_z.wait()

            def row(r, c2):
                for cc in range(D // 16):
                    sl = pl.ds(cc * 16, 16)
                    hat_v[r, sl] = W_ORI * z_v[r, sl] + W_MAP * rows_v[r, sl]
                return c2

            lax.fori_loop(0, CH, row, 0)
            pltpu.sync_copy(hat_v, zh_hbm.at[pl.ds(cb, CH)])
            return carry

        lax.fori_loop(0, NCH, chunk, 0)

    return body


def kernel(z, P):
    idx = _tc_argmin(z, P)
    z_hat, z_tilde = _sc_gather_blend()(z, P, idx)
    return (z_hat, z_tilde, idx)


# speedup vs baseline: 1.1911x; 1.0980x over previous
import functools

import jax
import jax.numpy as jnp
from jax import lax
from jax.experimental import pallas as pl
from jax.experimental.pallas import tpu as pltpu
from jax.experimental.pallas import tpu_sc as plsc

B = 16384
K = 512
D = 512
W_ORI = 0.7
W_MAP = 0.3


TC_BLK = 512
TC_GRID = B // TC_BLK


def _tc_argmin_body(z_ref, p_ref, pn_ref, idx_ref):
    z = z_ref[...]
    p = p_ref[...]
    scores = lax.dot_general(
        z, p, dimension_numbers=(((1,), (1,)), ((), ())),
        preferred_element_type=jnp.float32)
    zn = jnp.sum(z * z, axis=1, keepdims=True)
    norms = zn + pn_ref[...]
    dists = norms - 2.0 * scores
    m = jnp.min(dists, axis=1, keepdims=True)
    iota = lax.broadcasted_iota(jnp.int32, dists.shape, 1)
    cand = jnp.where(dists == m, iota, K)
    idx_ref[...] = jnp.min(cand, axis=1, keepdims=True)


def _tc_argmin(z, p):
    pn = jnp.sum(p * p, axis=1)[None, :]
    return pl.pallas_call(
        _tc_argmin_body,
        grid=(TC_GRID,),
        in_specs=[
            pl.BlockSpec((TC_BLK, D), lambda i: (i, 0)),
            pl.BlockSpec((K, D), lambda i: (0, 0)),
            pl.BlockSpec((1, K), lambda i: (0, 0)),
        ],
        out_specs=pl.BlockSpec((TC_BLK, 1), lambda i: (i, 0)),
        out_shape=jax.ShapeDtypeStruct((B, 1), jnp.int32),
    )(z, p, pn)


def _tc_blend_body(z_ref, p_ref, idx_ref, out_ref):
    idx = idx_ref[...]
    iota = lax.broadcasted_iota(jnp.int32, (TC_BLK, K), 1)
    onehot = (iota == idx).astype(jnp.float32)
    zt = lax.dot_general(
        onehot, p_ref[...], dimension_numbers=(((1,), (0,)), ((), ())),
        preferred_element_type=jnp.float32)
    out_ref[...] = W_ORI * z_ref[...] + W_MAP * zt


def _tc_blend(z, p, idx2d):
    return pl.pallas_call(
        _tc_blend_body,
        grid=(TC_GRID,),
        in_specs=[
            pl.BlockSpec((TC_BLK, D), lambda i: (i, 0)),
            pl.BlockSpec((K, D), lambda i: (0, 0)),
            pl.BlockSpec((TC_BLK, 1), lambda i: (i, 0)),
        ],
        out_specs=pl.BlockSpec((TC_BLK, D), lambda i: (i, 0)),
        out_shape=jax.ShapeDtypeStruct((B, D), jnp.float32),
    )(z, p, idx2d)


NC = 2
NS = 16
NW = NC * NS
ROWS_W = B // NW
CH = 64
NCH = ROWS_W // CH


@functools.cache
def _sc_gather():
    mesh = plsc.VectorSubcoreMesh(core_axis_name="c", subcore_axis_name="s")

    @functools.partial(
        pl.kernel,
        out_type=jax.ShapeDtypeStruct((B, D), jnp.float32),
        mesh=mesh,
        scratch_types=[
            pltpu.VMEM((CH,), jnp.int32),
            pltpu.VMEM((CH,), jnp.int32),
            pltpu.VMEM((CH, D), jnp.float32),
            pltpu.VMEM((CH, D), jnp.float32),
            pltpu.SemaphoreType.DMA,
            pltpu.SemaphoreType.DMA,
            pltpu.SemaphoreType.DMA,
            pltpu.SemaphoreType.DMA,
        ],
    )
    def body(p_hbm, idx_hbm, zt_hbm,
             idx0, idx1, buf0, buf1, sg0, sg1, sw0, sw1):
        wid = lax.axis_index("s") * NC + lax.axis_index("c")
        base = wid * ROWS_W
        idx_r = (idx0, idx1)
        bufs = (buf0, buf1)
        sg = (sg0, sg1)
        sw = (sw0, sw1)

        pltpu.sync_copy(idx_hbm.at[pl.ds(base, CH)], idx0)
        gathers = [pltpu.async_copy(p_hbm.at[idx0], buf0, sg0), None]
        writes = [None, None]
        for ci in range(NCH):
            cur = ci % 2
            nxt = (ci + 1) % 2
            if ci + 1 < NCH:
                pltpu.sync_copy(
                    idx_hbm.at[pl.ds(base + (ci + 1) * CH, CH)], idx_r[nxt])
                if writes[nxt] is not None:
                    writes[nxt].wait()
                gathers[nxt] = pltpu.async_copy(
                    p_hbm.at[idx_r[nxt]], bufs[nxt], sg[nxt])
            gathers[cur].wait()
            writes[cur] = pltpu.async_copy(
                bufs[cur], zt_hbm.at[pl.ds(base + ci * CH, CH)], sw[cur])
        writes[0].wait()
        writes[1].wait()

    return body


def kernel(z, P):
    idx2d = _tc_argmin(z, P)
    idx = idx2d.reshape(B)
    z_tilde = _sc_gather()(P, idx)
    z_hat = _tc_blend(z, P, idx2d)
    return (z_hat, z_tilde, idx)


# speedup vs baseline: 1.3496x; 1.1331x over previous
import functools

import jax
import jax.numpy as jnp
from jax import lax
from jax.experimental import pallas as pl
from jax.experimental.pallas import tpu as pltpu
from jax.experimental.pallas import tpu_sc as plsc

B = 16384
K = 512
D = 512
W_ORI = 0.7
W_MAP = 0.3


TC_BLK = 512
TC_GRID = B // TC_BLK


def _tc_argmin_body(z_ref, p_ref, idx_ref):
    z = z_ref[...]
    p = p_ref[...]
    scores_t = lax.dot_general(
        p, z, dimension_numbers=(((1,), (1,)), ((), ())),
        preferred_element_type=jnp.float32)
    zn = jnp.sum(z * z, axis=1, keepdims=True)
    pn = jnp.sum(p * p, axis=1, keepdims=True)
    zn_row = lax.transpose(zn, (1, 0))
    dists_t = (zn_row + pn) - 2.0 * scores_t
    m = jnp.min(dists_t, axis=0, keepdims=True)
    iota = lax.broadcasted_iota(jnp.int32, dists_t.shape, 0)
    cand = jnp.where(dists_t == m, iota, K)
    idx_ref[0, :, :] = jnp.min(cand, axis=0, keepdims=True)


def _tc_argmin(z, p):
    out = pl.pallas_call(
        _tc_argmin_body,
        grid=(TC_GRID,),
        in_specs=[
            pl.BlockSpec((TC_BLK, D), lambda i: (i, 0)),
            pl.BlockSpec((K, D), lambda i: (0, 0)),
        ],
        out_specs=pl.BlockSpec((1, 1, TC_BLK), lambda i: (i, 0, 0)),
        out_shape=jax.ShapeDtypeStruct((TC_GRID, 1, TC_BLK), jnp.int32),
    )(z, p)
    return out


def _tc_blend_body(z_ref, p_ref, idx_ref, out_ref):
    idx_row = idx_ref[0]
    iota = lax.broadcasted_iota(jnp.int32, (K, TC_BLK), 0)
    onehot_t = (iota == idx_row).astype(jnp.float32)
    zt = lax.dot_general(
        onehot_t, p_ref[...], dimension_numbers=(((0,), (0,)), ((), ())),
        preferred_element_type=jnp.float32)
    out_ref[...] = W_ORI * z_ref[...] + W_MAP * zt


def _tc_blend(z, p, idx3):
    return pl.pallas_call(
        _tc_blend_body,
        grid=(TC_GRID,),
        in_specs=[
            pl.BlockSpec((TC_BLK, D), lambda i: (i, 0)),
            pl.BlockSpec((K, D), lambda i: (0, 0)),
            pl.BlockSpec((1, 1, TC_BLK), lambda i: (i, 0, 0)),
        ],
        out_specs=pl.BlockSpec((TC_BLK, D), lambda i: (i, 0)),
        out_shape=jax.ShapeDtypeStruct((B, D), jnp.float32),
    )(z, p, idx3)


NC = 2
NS = 16
NW = NC * NS
ROWS_W = B // NW
CH = 64
NCH = ROWS_W // CH


@functools.cache
def _sc_gather():
    mesh = plsc.VectorSubcoreMesh(core_axis_name="c", subcore_axis_name="s")

    @functools.partial(
        pl.kernel,
        out_type=jax.ShapeDtypeStruct((B, D), jnp.float32),
        mesh=mesh,
        scratch_types=[
            pltpu.VMEM((CH,), jnp.int32),
            pltpu.VMEM((CH,), jnp.int32),
            pltpu.VMEM((CH, D), jnp.float32),
            pltpu.VMEM((CH, D), jnp.float32),
            pltpu.SemaphoreType.DMA,
            pltpu.SemaphoreType.DMA,
            pltpu.SemaphoreType.DMA,
            pltpu.SemaphoreType.DMA,
        ],
    )
    def body(p_hbm, idx_hbm, zt_hbm,
             idx0, idx1, buf0, buf1, sg0, sg1, sw0, sw1):
        wid = lax.axis_index("s") * NC + lax.axis_index("c")
        base = wid * ROWS_W
        idx_r = (idx0, idx1)
        bufs = (buf0, buf1)
        sg = (sg0, sg1)
        sw = (sw0, sw1)

        pltpu.sync_copy(idx_hbm.at[pl.ds(base, CH)], idx0)
        gathers = [pltpu.async_copy(p_hbm.at[idx0], buf0, sg0), None]
        writes = [None, None]
        for ci in range(NCH):
            cur = ci % 2
            nxt = (ci + 1) % 2
            if ci + 1 < NCH:
                pltpu.sync_copy(
                    idx_hbm.at[pl.ds(base + (ci + 1) * CH, CH)], idx_r[nxt])
                if writes[nxt] is not None:
                    writes[nxt].wait()
                gathers[nxt] = pltpu.async_copy(
                    p_hbm.at[idx_r[nxt]], bufs[nxt], sg[nxt])
            gathers[cur].wait()
            writes[cur] = pltpu.async_copy(
                bufs[cur], zt_hbm.at[pl.ds(base + ci * CH, CH)], sw[cur])
        writes[0].wait()
        writes[1].wait()

    return body


def kernel(z, P):
    idx3 = _tc_argmin(z, P)
    idx = idx3.reshape(B)
    z_tilde = _sc_gather()(P, idx)
    z_hat = _tc_blend(z, P, idx3)
    return (z_hat, z_tilde, idx)


# speedup vs baseline: 1.6405x; 1.2155x over previous
import functools

import jax
import jax.numpy as jnp
from jax import lax
from jax.experimental import pallas as pl
from jax.experimental.pallas import tpu as pltpu
from jax.experimental.pallas import tpu_sc as plsc

B = 16384
K = 512
D = 512
W_ORI = 0.7
W_MAP = 0.3


TC_BLK = 2048
TC_GRID = B // TC_BLK


def _tc_argmin_body(z_ref, p_ref, idx_ref):
    z = z_ref[...]
    p = p_ref[...]
    scores_t = lax.dot_general(
        p, z, dimension_numbers=(((1,), (1,)), ((), ())),
        preferred_element_type=jnp.float32)
    zn = jnp.sum(z * z, axis=1, keepdims=True)
    pn = jnp.sum(p * p, axis=1, keepdims=True)
    zn_row = lax.transpose(zn, (1, 0))
    dists_t = (zn_row + pn) - 2.0 * scores_t
    m = jnp.min(dists_t, axis=0, keepdims=True)
    iota = lax.broadcasted_iota(jnp.int32, dists_t.shape, 0)
    cand = jnp.where(dists_t == m, iota, K)
    idx_ref[0, :, :] = jnp.min(cand, axis=0, keepdims=True)


def _tc_argmin(z, p):
    out = pl.pallas_call(
        _tc_argmin_body,
        grid=(TC_GRID,),
        in_specs=[
            pl.BlockSpec((TC_BLK, D), lambda i: (i, 0)),
            pl.BlockSpec((K, D), lambda i: (0, 0)),
        ],
        out_specs=pl.BlockSpec((1, 1, TC_BLK), lambda i: (i, 0, 0)),
        out_shape=jax.ShapeDtypeStruct((TC_GRID, 1, TC_BLK), jnp.int32),
    )(z, p)
    return out


def _tc_blend_body(z_ref, p_ref, idx_ref, out_ref):
    idx_row = idx_ref[0]
    iota = lax.broadcasted_iota(jnp.int32, (K, TC_BLK), 0)
    onehot_t = (iota == idx_row).astype(jnp.float32)
    zt = lax.dot_general(
        onehot_t, p_ref[...], dimension_numbers=(((0,), (0,)), ((), ())),
        preferred_element_type=jnp.float32)
    out_ref[...] = W_ORI * z_ref[...] + W_MAP * zt


def _tc_blend(z, p, idx3):
    return pl.pallas_call(
        _tc_blend_body,
        grid=(TC_GRID,),
        in_specs=[
            pl.BlockSpec((TC_BLK, D), lambda i: (i, 0)),
            pl.BlockSpec((K, D), lambda i: (0, 0)),
            pl.BlockSpec((1, 1, TC_BLK), lambda i: (i, 0, 0)),
        ],
        out_specs=pl.BlockSpec((TC_BLK, D), lambda i: (i, 0)),
        out_shape=jax.ShapeDtypeStruct((B, D), jnp.float32),
    )(z, p, idx3)


NC = 2
NS = 16
NW = NC * NS
ROWS_W = B // NW
CH = 64
NCH = ROWS_W // CH


@functools.cache
def _sc_gather():
    mesh = plsc.VectorSubcoreMesh(core_axis_name="c", subcore_axis_name="s")

    @functools.partial(
        pl.kernel,
        out_type=jax.ShapeDtypeStruct((B, D), jnp.float32),
        mesh=mesh,
        scratch_types=[
            pltpu.VMEM((CH,), jnp.int32),
            pltpu.VMEM((CH,), jnp.int32),
            pltpu.VMEM((CH, D), jnp.float32),
            pltpu.VMEM((CH, D), jnp.float32),
            pltpu.SemaphoreType.DMA,
            pltpu.SemaphoreType.DMA,
            pltpu.SemaphoreType.DMA,
            pltpu.SemaphoreType.DMA,
        ],
    )
    def body(p_hbm, idx_hbm, zt_hbm,
             idx0, idx1, buf0, buf1, sg0, sg1, sw0, sw1):
        wid = lax.axis_index("s") * NC + lax.axis_index("c")
        base = wid * ROWS_W
        idx_r = (idx0, idx1)
        bufs = (buf0, buf1)
        sg = (sg0, sg1)
        sw = (sw0, sw1)

        pltpu.sync_copy(idx_hbm.at[pl.ds(base, CH)], idx0)
        gathers = [pltpu.async_copy(p_hbm.at[idx0], buf0, sg0), None]
        writes = [None, None]
        for ci in range(NCH):
            cur = ci % 2
            nxt = (ci + 1) % 2
            if ci + 1 < NCH:
                pltpu.sync_copy(
                    idx_hbm.at[pl.ds(base + (ci + 1) * CH, CH)], idx_r[nxt])
                if writes[nxt] is not None:
                    writes[nxt].wait()
                gathers[nxt] = pltpu.async_copy(
                    p_hbm.at[idx_r[nxt]], bufs[nxt], sg[nxt])
            gathers[cur].wait()
            writes[cur] = pltpu.async_copy(
                bufs[cur], zt_hbm.at[pl.ds(base + ci * CH, CH)], sw[cur])
        writes[0].wait()
        writes[1].wait()

    return body


def kernel(z, P):
    idx3 = _tc_argmin(z, P)
    idx = idx3.reshape(B)
    z_tilde = _sc_gather()(P, idx)
    z_hat = _tc_blend(z, P, idx3)
    return (z_hat, z_tilde, idx)


# speedup vs baseline: 1.6488x; 1.0050x over previous
import functools

import jax
import jax.numpy as jnp
from jax import lax
from jax.experimental import pallas as pl
from jax.experimental.pallas import tpu as pltpu
from jax.experimental.pallas import tpu_sc as plsc

B = 16384
K = 512
D = 512
W_ORI = 0.7
W_MAP = 0.3


TC_BLK = 4096
TC_GRID = B // TC_BLK


def _tc_argmin_body(z_ref, p_ref, idx_ref):
    z = z_ref[...]
    p = p_ref[...]
    scores_t = lax.dot_general(
        p, z, dimension_numbers=(((1,), (1,)), ((), ())),
        preferred_element_type=jnp.float32)
    zn = jnp.sum(z * z, axis=1, keepdims=True)
    pn = jnp.sum(p * p, axis=1, keepdims=True)
    zn_row = lax.transpose(zn, (1, 0))
    dists_t = (zn_row + pn) - 2.0 * scores_t
    m = jnp.min(dists_t, axis=0, keepdims=True)
    iota = lax.broadcasted_iota(jnp.int32, dists_t.shape, 0)
    cand = jnp.where(dists_t == m, iota, K)
    idx_ref[0, :, :] = jnp.min(cand, axis=0, keepdims=True)


def _tc_argmin(z, p):
    out = pl.pallas_call(
        _tc_argmin_body,
        grid=(TC_GRID,),
        in_specs=[
            pl.BlockSpec((TC_BLK, D), lambda i: (i, 0)),
            pl.BlockSpec((K, D), lambda i: (0, 0)),
        ],
        out_specs=pl.BlockSpec((1, 1, TC_BLK), lambda i: (i, 0, 0)),
        out_shape=jax.ShapeDtypeStruct((TC_GRID, 1, TC_BLK), jnp.int32),
    )(z, p)
    return out


def _tc_blend_body(z_ref, p_ref, idx_ref, out_ref):
    idx_row = idx_ref[0]
    iota = lax.broadcasted_iota(jnp.int32, (K, TC_BLK), 0)
    onehot_t = (iota == idx_row).astype(jnp.float32)
    zt = lax.dot_general(
        onehot_t, p_ref[...], dimension_numbers=(((0,), (0,)), ((), ())),
        preferred_element_type=jnp.float32)
    out_ref[...] = W_ORI * z_ref[...] + W_MAP * zt


def _tc_blend(z, p, idx3):
    return pl.pallas_call(
        _tc_blend_body,
        grid=(TC_GRID,),
        in_specs=[
            pl.BlockSpec((TC_BLK, D), lambda i: (i, 0)),
            pl.BlockSpec((K, D), lambda i: (0, 0)),
            pl.BlockSpec((1, 1, TC_BLK), lambda i: (i, 0, 0)),
        ],
        out_specs=pl.BlockSpec((TC_BLK, D), lambda i: (i, 0)),
        out_shape=jax.ShapeDtypeStruct((B, D), jnp.float32),
    )(z, p, idx3)


NC = 2
NS = 16
NW = NC * NS
ROWS_W = B // NW
CH = 64
NCH = ROWS_W // CH


@functools.cache
def _sc_gather():
    mesh = plsc.VectorSubcoreMesh(core_axis_name="c", subcore_axis_name="s")

    @functools.partial(
        pl.kernel,
        out_type=jax.ShapeDtypeStruct((B, D), jnp.float32),
        mesh=mesh,
        scratch_types=[
            pltpu.VMEM((CH,), jnp.int32),
            pltpu.VMEM((CH,), jnp.int32),
            pltpu.VMEM((CH, D), jnp.float32),
            pltpu.VMEM((CH, D), jnp.float32),
            pltpu.SemaphoreType.DMA,
            pltpu.SemaphoreType.DMA,
            pltpu.SemaphoreType.DMA,
            pltpu.SemaphoreType.DMA,
        ],
    )
    def body(p_hbm, idx_hbm, zt_hbm,
             idx0, idx1, buf0, buf1, sg0, sg1, sw0, sw1):
        wid = lax.axis_index("s") * NC + lax.axis_index("c")
        base = wid * ROWS_W
        idx_r = (idx0, idx1)
        bufs = (buf0, buf1)
        sg = (sg0, sg1)
        sw = (sw0, sw1)

        pltpu.sync_copy(idx_hbm.at[pl.ds(base, CH)], idx0)
        gathers = [pltpu.async_copy(p_hbm.at[idx0], buf0, sg0), None]
        writes = [None, None]
        for ci in range(NCH):
            cur = ci % 2
            nxt = (ci + 1) % 2
            if ci + 1 < NCH:
                pltpu.sync_copy(
                    idx_hbm.at[pl.ds(base + (ci + 1) * CH, CH)], idx_r[nxt])
                if writes[nxt] is not None:
                    writes[nxt].wait()
                gathers[nxt] = pltpu.async_copy(
                    p_hbm.at[idx_r[nxt]], bufs[nxt], sg[nxt])
            gathers[cur].wait()
            writes[cur] = pltpu.async_copy(
                bufs[cur], zt_hbm.at[pl.ds(base + ci * CH, CH)], sw[cur])
        writes[0].wait()
        writes[1].wait()

    return body


def kernel(z, P):
    idx3 = _tc_argmin(z, P)
    idx = idx3.reshape(B)
    z_tilde = _sc_gather()(P, idx)
    z_hat = _tc_blend(z, P, idx3)
    return (z_hat, z_tilde, idx)
